# baseline (speedup 1.0000x reference)
import functools

import jax
import jax.numpy as jnp
from jax import lax
from jax.experimental import pallas as pl
from jax.experimental.pallas import tpu as pltpu
from jax.experimental.pallas import tpu_sc as plsc

N = 10000
E = 320000
F = 128
EF = 16
H = 128
D = 5
NG = 256

NC = 2
NS = 16
NW = NC * NS
EPW = E // NW
SLAB = 40
NSLAB = EPW // SLAB
CS = 50
NCH = NSLAB // CS
RPT = 624
RPT_LAST = N - (NS - 1) * RPT

MB = 2000
EB = 1600

_f32 = jnp.float32
_bf16 = jnp.bfloat16
_HI = lax.Precision.HIGHEST
_HIMASK = -65536
HW = H // 2


def _pack_bf16_pair(x):
    lo = lax.bitcast_convert_type(x[:, :HW].astype(_bf16), jnp.uint16)
    hi = lax.bitcast_convert_type(x[:, HW:].astype(_bf16), jnp.uint16)
    w = lo.astype(jnp.uint32) | (hi.astype(jnp.uint32) << 16)
    return lax.bitcast_convert_type(w, jnp.int32)


def _init_body(x_ref, pnw_ref, pnb_ref, wt0_ref, wb0_ref, peb_ref, mb0_ref,
               h_ref, hw_ref):
    h = jnp.maximum(
        jnp.dot(x_ref[...], pnw_ref[...], preferred_element_type=_f32, precision=_HI)
        + pnb_ref[...], 0.0)
    h_ref[...] = h
    bp = jnp.dot(peb_ref[...], wb0_ref[...], preferred_element_type=_f32, precision=_HI) \
        + mb0_ref[...]
    hw_ref[...] = jnp.dot(h, wt0_ref[...], preferred_element_type=_f32,
                          precision=_HI) + bp


def _ew_body(ea_ref, pew_ref, wb_ref, out_ref):
    wall = jnp.dot(pew_ref[...], wb_ref[0], preferred_element_type=_f32, precision=_HI)
    out_ref[...] = _pack_bf16_pair(
        jnp.dot(ea_ref[...], wall, preferred_element_type=_f32, precision=_HI))


def _upd_body(h_ref, agg_ref, w1h_ref, w1a_ref, b1_ref, w2_ref, b2_ref,
              wtn_ref, wbn_ref, peb_ref, mbn_ref, hn_ref, hwn_ref):
    agg = agg_ref[0] + agg_ref[1]
    u = jnp.maximum(
        jnp.dot(h_ref[...], w1h_ref[...], preferred_element_type=_f32, precision=_HI)
        + jnp.dot(agg, w1a_ref[...], preferred_element_type=_f32, precision=_HI)
        + b1_ref[...], 0.0)
    hn = jnp.dot(u, w2_ref[...], preferred_element_type=_f32, precision=_HI) + b2_ref[...]
    hn = jnp.maximum(hn, 0.0)
    hn_ref[...] = hn
    bp = jnp.dot(peb_ref[...], wbn_ref[...], preferred_element_type=_f32, precision=_HI) \
        + mbn_ref[...]
    hwn_ref[...] = jnp.dot(hn, wtn_ref[...], preferred_element_type=_f32,
                           precision=_HI) + bp


def _upd_last_body(h_ref, agg_ref, w1h_ref, w1a_ref, b1_ref, w2_ref, b2_ref,
                   hn_ref):
    agg = agg_ref[0] + agg_ref[1]
    u = jnp.maximum(
        jnp.dot(h_ref[...], w1h_ref[...], preferred_element_type=_f32, precision=_HI)
        + jnp.dot(agg, w1a_ref[...], preferred_element_type=_f32, precision=_HI)
        + b1_ref[...], 0.0)
    hn_ref[...] = jnp.dot(u, w2_ref[...], preferred_element_type=_f32, precision=_HI) \
        + b2_ref[...]


def _prelu(x, a):
    return jnp.maximum(x, 0.0) + a * jnp.minimum(x, 0.0)


def _readout_body(h_ref, b_ref, spw_ref, spb_ref, spa_ref,
                  w1_ref, b1_ref, a1_ref, w2_ref, b2_ref, a2_ref,
                  w3_ref, b3_ref, out_ref, acc_ref):
    m = pl.program_id(0)
    onehot = (b_ref[...] == lax.broadcasted_iota(jnp.int32, (1, NG), 1)
              ).astype(_f32)
    part = lax.dot_general(onehot, h_ref[...], (((0,), (0,)), ((), ())),
                           preferred_element_type=_f32, precision=_HI)

    @pl.when(m == 0)
    def _():
        acc_ref[...] = part

    @pl.when(m > 0)
    def _():
        acc_ref[...] += part

    @pl.when(m == N // MB - 1)
    def _():
        r = jnp.dot(acc_ref[...], spw_ref[...], preferred_element_type=_f32, precision=_HI) \
            + spb_ref[...]
        r = _prelu(r, spa_ref[0, 0])
        p = jnp.dot(r, w1_ref[...], preferred_element_type=_f32, precision=_HI) + b1_ref[...]
        p = _prelu(p, a1_ref[0, 0])
        p = jnp.dot(p, w2_ref[...], preferred_element_type=_f32, precision=_HI) + b2_ref[...]
        p = _prelu(p, a2_ref[0, 0])
        out_ref[...] = jnp.dot(p, w3_ref[...], preferred_element_type=_f32, precision=_HI) \
            + b3_ref[...]


def _sc_edge_body(i, hw_hbm, ew_hbm, src_hbm, dst_hbm, z_hbm, agg_hbm,
                  src_v, dst_v, gbuf0, gbuf1, ebuf0, ebuf1, mbuf, agg_sh,
                  gsem0, gsem1, esem0, esem1):
    c = lax.axis_index("c")
    s = lax.axis_index("s")
    wid = s * NC + c
    ebase = wid * EPW

    @pl.when(s < NS - 1)
    def _():
        pltpu.sync_copy(z_hbm.at[pl.ds(s * RPT, RPT)],
                        agg_sh.at[pl.ds(s * RPT, RPT)])

    @pl.when(s == NS - 1)
    def _():
        pltpu.sync_copy(z_hbm.at[pl.ds((NS - 1) * RPT, RPT_LAST)],
                        agg_sh.at[pl.ds((NS - 1) * RPT, RPT_LAST)])

    plsc.subcore_barrier()

    def start(cc, j, gb, eb, gs, es):
        pltpu.async_copy(hw_hbm.at[src_v.at[j]], gb, gs)
        pltpu.async_copy(
            ew_hbm.at[pl.ds(i * E + ebase + (cc * CS + j) * SLAB, SLAB)],
            eb, es)

    def finish(cc, j, gb, eb, gs, es):
        pltpu.make_async_copy(hw_hbm.at[src_v.at[j]], gb, gs).wait()
        pltpu.make_async_copy(
            ew_hbm.at[pl.ds(i * E + ebase + (cc * CS + j) * SLAB, SLAB)],
            eb, es).wait()

        def row(r, c2):
            for k in range(HW // 16):
                ei = eb[r, pl.ds(16 * k, 16)]
                lo = jnp.maximum(
                    gb[r, pl.ds(16 * k, 16)]
                    + lax.bitcast_convert_type(ei << 16, _f32), 0.0)
                hi = jnp.maximum(
                    gb[r, pl.ds(HW + 16 * k, 16)]
                    + lax.bitcast_convert_type(ei & _HIMASK, _f32), 0.0)
                mbuf[r, pl.ds(16 * k, 16)] = lo
                mbuf[r, pl.ds(HW + 16 * k, 16)] = hi
            return c2

        lax.fori_loop(0, SLAB, row, 0)
        pltpu.sync_copy(mbuf, agg_sh.at[dst_v.at[j]], add=True)

    def chunk(cc, carry):
        pltpu.sync_copy(src_hbm.at[wid].at[cc], src_v)
        pltpu.sync_copy(dst_hbm.at[wid].at[cc], dst_v)
        start(cc, 0, gbuf0, ebuf0, gsem0, esem0)

        def pair(jj, c1):
            j0 = 2 * jj
            start(cc, j0 + 1, gbuf1, ebuf1, gsem1, esem1)
            finish(cc, j0, gbuf0, ebuf0, gsem0, esem0)

            @pl.when(jj < CS // 2 - 1)
            def _():
                start(cc, j0 + 2, gbuf0, ebuf0, gsem0, esem0)

            finish(cc, j0 + 1, gbuf1, ebuf1, gsem1, esem1)
            return c1

        lax.fori_loop(0, CS // 2, pair, 0)
        return carry

    lax.fori_loop(0, NCH, chunk, 0)
    plsc.subcore_barrier()

    @pl.when(s < NS - 1)
    def _():
        pltpu.sync_copy(agg_sh.at[pl.ds(s * RPT, RPT)],
                        agg_hbm.at[c].at[pl.ds(s * RPT, RPT)])

    @pl.when(s == NS - 1)
    def _():
        pltpu.sync_copy(agg_sh.at[pl.ds((NS - 1) * RPT, RPT_LAST)],
                        agg_hbm.at[c].at[pl.ds((NS - 1) * RPT, RPT_LAST)])


def _sc_aggregate(i, hw, ew, srcg, dstg, zeros):
    mesh = plsc.VectorSubcoreMesh(core_axis_name="c", subcore_axis_name="s")
    return pl.kernel(
        functools.partial(_sc_edge_body, i),
        out_type=jax.ShapeDtypeStruct((NC, N, H), _f32),
        mesh=mesh,
        scratch_types=[
            pltpu.VMEM((CS, SLAB), jnp.int32),
            pltpu.VMEM((CS, SLAB), jnp.int32),
            pltpu.VMEM((SLAB, H), _f32),
            pltpu.VMEM((SLAB, H), _f32),
            pltpu.VMEM((SLAB, HW), jnp.int32),
            pltpu.VMEM((SLAB, HW), jnp.int32),
            pltpu.VMEM((SLAB, H), _f32),
            pltpu.VMEM_SHARED((N, H), _f32),
            pltpu.SemaphoreType.DMA,
            pltpu.SemaphoreType.DMA,
            pltpu.SemaphoreType.DMA,
            pltpu.SemaphoreType.DMA,
        ],
    )(hw, ew, srcg, dstg, zeros)


def _full(shape):
    return pl.BlockSpec(shape, lambda *_: tuple(0 for _ in shape))


def kernel(x, edge_index, edge_attr, batch,
           pn_W, pn_b, pe_W, pe_b,
           msg_W, msg_b, upd_W1, upd_b1, upd_W2, upd_b2,
           sp_W, sp_b, sp_a,
           pr_W1, pr_b1, pr_a1, pr_W2, pr_b2, pr_a2, pr_W3, pr_b3):
    srcg = edge_index[0].reshape(NW, NCH, CS, SLAB)
    dstg = edge_index[1].reshape(NW, NCH, CS, SLAB)
    zeros = jnp.zeros((N, H), _f32)

    wt = msg_W[:, :H, :]
    wb = msg_W[:, H:, :]
    w1h = upd_W1[:, :H, :]
    w1a = upd_W1[:, H:, :]
    pn_b2 = pn_b.reshape(1, H)
    pe_b2 = pe_b.reshape(1, H)
    msg_b2 = msg_b.reshape(D, 1, H)
    b1_2 = upd_b1.reshape(D, 1, 2 * H)
    b2_2 = upd_b2.reshape(D, 1, H)

    nm = N // MB

    h, hw = pl.pallas_call(
        _init_body,
        grid=(nm,),
        in_specs=[
            pl.BlockSpec((MB, F), lambda m: (m, 0)),
            _full((F, H)), _full((1, H)), _full((H, H)), _full((H, H)),
            _full((1, H)), _full((1, H)),
        ],
        out_specs=[pl.BlockSpec((MB, H), lambda m: (m, 0)),
                   pl.BlockSpec((MB, H), lambda m: (m, 0))],
        out_shape=[jax.ShapeDtypeStruct((N, H), _f32),
                   jax.ShapeDtypeStruct((N, H), _f32)],
    )(x, pn_W, pn_b2, wt[0], wb[0], pe_b2, msg_b2[0])

    ne = E // EB
    ew = pl.pallas_call(
        _ew_body,
        grid=(ne, D),
        in_specs=[
            pl.BlockSpec((EB, EF), lambda e, i: (e, 0)),
            pl.BlockSpec((EF, H), lambda e, i: (0, 0)),
            pl.BlockSpec((1, H, H), lambda e, i: (i, 0, 0)),
        ],
        out_specs=pl.BlockSpec((EB, HW), lambda e, i: (i * ne + e, 0)),
        out_shape=jax.ShapeDtypeStruct((D * E, HW), jnp.int32),
    )(edge_attr, pe_W, wb)

    for i in range(D):
        aggp = _sc_aggregate(i, hw, ew, srcg, dstg, zeros)
        if i < D - 1:
            h, hw = pl.pallas_call(
                _upd_body,
                grid=(nm,),
                in_specs=[
                    pl.BlockSpec((MB, H), lambda m: (m, 0)),
                    pl.BlockSpec((NC, MB, H), lambda m: (0, m, 0)),
                    _full((H, 2 * H)), _full((H, 2 * H)), _full((1, 2 * H)),
                    _full((2 * H, H)), _full((1, H)),
                    _full((H, H)), _full((H, H)), _full((1, H)),
                    _full((1, H)),
                ],
                out_specs=[pl.BlockSpec((MB, H), lambda m: (m, 0)),
                           pl.BlockSpec((MB, H), lambda m: (m, 0))],
                out_shape=[jax.ShapeDtypeStruct((N, H), _f32),
                           jax.ShapeDtypeStruct((N, H), _f32)],
            )(h, aggp, w1h[i], w1a[i], b1_2[i], upd_W2[i], b2_2[i],
              wt[i + 1], wb[i + 1], pe_b2, msg_b2[i + 1])
        else:
            h = pl.pallas_call(
                _upd_last_body,
                grid=(nm,),
                in_specs=[
                    pl.BlockSpec((MB, H), lambda m: (m, 0)),
                    pl.BlockSpec((NC, MB, H), lambda m: (0, m, 0)),
                    _full((H, 2 * H)), _full((H, 2 * H)), _full((1, 2 * H)),
                    _full((2 * H, H)), _full((1, H)),
                ],
                out_specs=pl.BlockSpec((MB, H), lambda m: (m, 0)),
                out_shape=jax.ShapeDtypeStruct((N, H), _f32),
            )(h, aggp, w1h[i], w1a[i], b1_2[i], upd_W2[i], b2_2[i])

    w3p = jnp.pad(pr_W3, ((0, 0), (0, F - pr_W3.shape[1])))
    out_p = pl.pallas_call(
        _readout_body,
        grid=(nm,),
        in_specs=[
            pl.BlockSpec((MB, H), lambda m: (m, 0)),
            pl.BlockSpec((MB, 1), lambda m: (m, 0)),
            _full((H, 256)), _full((1, 256)), _full((1, 1)),
            _full((256, 256)), _full((1, 256)), _full((1, 1)),
            _full((256, 256)), _full((1, 256)), _full((1, 1)),
            _full((256, F)), _full((1, 1)),
        ],
        out_specs=pl.BlockSpec((NG, F), lambda m: (0, 0)),
        out_shape=jax.ShapeDtypeStruct((NG, F), _f32),
        scratch_shapes=[pltpu.VMEM((NG, H), _f32)],
    )(h, batch.reshape(N, 1), sp_W, sp_b.reshape(1, 256),
      sp_a.reshape(1, 1).astype(_f32),
      pr_W1, pr_b1.reshape(1, 256), pr_a1.reshape(1, 1).astype(_f32),
      pr_W2, pr_b2.reshape(1, 256), pr_a2.reshape(1, 1).astype(_f32),
      w3p, pr_b3.reshape(1, 1))
    return out_p[:, :pr_W3.shape[1]]


# speedup vs baseline: 1.5083x; 1.5083x over previous
import functools

import jax
import jax.numpy as jnp
from jax import lax
from jax.experimental import pallas as pl
from jax.experimental.pallas import tpu as pltpu
from jax.experimental.pallas import tpu_sc as plsc

N = 10000
E = 320000
F = 128
EF = 16
H = 128
D = 5
NG = 256

NC = 2
NS = 16
NW = NC * NS
EPW = E // NW
SLAB = 40
NSLAB = EPW // SLAB
CS = 50
NCH = NSLAB // CS
RPT = 624
RPT_LAST = N - (NS - 1) * RPT

MB = 2000
EB = 6400

_f32 = jnp.float32
_bf16 = jnp.bfloat16
_HI = lax.Precision.HIGHEST
_HIMASK = -65536
HW = H // 2


def _pack_bf16_pair(x):
    lo = lax.bitcast_convert_type(x[:, :HW].astype(_bf16), jnp.uint16)
    hi = lax.bitcast_convert_type(x[:, HW:].astype(_bf16), jnp.uint16)
    w = lo.astype(jnp.uint32) | (hi.astype(jnp.uint32) << 16)
    return lax.bitcast_convert_type(w, jnp.int32)


def _init_body(x_ref, pnw_ref, pnb_ref, wt0_ref, wb0_ref, peb_ref, mb0_ref,
               h_ref, hw_ref):
    h = jnp.maximum(
        jnp.dot(x_ref[...], pnw_ref[...], preferred_element_type=_f32, precision=_HI)
        + pnb_ref[...], 0.0)
    h_ref[...] = h
    bp = jnp.dot(peb_ref[...], wb0_ref[...], preferred_element_type=_f32, precision=_HI) \
        + mb0_ref[...]
    hw_ref[...] = jnp.dot(h, wt0_ref[...], preferred_element_type=_f32,
                          precision=_HI) + bp


def _ew_body(ea_ref, pew_ref, wb_ref, o0, o1, o2, o3, o4):
    ea = ea_ref[...]
    for i, o in enumerate((o0, o1, o2, o3, o4)):
        wall = jnp.dot(pew_ref[...], wb_ref[i],
                       preferred_element_type=_f32, precision=_HI)
        o[...] = _pack_bf16_pair(jnp.dot(ea, wall, preferred_element_type=_f32))


def _upd_body(h_ref, agg_ref, w1h_ref, w1a_ref, b1_ref, w2_ref, b2_ref,
              wtn_ref, wbn_ref, peb_ref, mbn_ref, hn_ref, hwn_ref):
    agg = agg_ref[0] + agg_ref[1]
    u = jnp.maximum(
        jnp.dot(h_ref[...], w1h_ref[...], preferred_element_type=_f32, precision=_HI)
        + jnp.dot(agg, w1a_ref[...], preferred_element_type=_f32, precision=_HI)
        + b1_ref[...], 0.0)
    hn = jnp.dot(u, w2_ref[...], preferred_element_type=_f32, precision=_HI) + b2_ref[...]
    hn = jnp.maximum(hn, 0.0)
    hn_ref[...] = hn
    bp = jnp.dot(peb_ref[...], wbn_ref[...], preferred_element_type=_f32, precision=_HI) \
        + mbn_ref[...]
    hwn_ref[...] = jnp.dot(hn, wtn_ref[...], preferred_element_type=_f32,
                           precision=_HI) + bp


def _upd_last_body(h_ref, agg_ref, w1h_ref, w1a_ref, b1_ref, w2_ref, b2_ref,
                   hn_ref):
    agg = agg_ref[0] + agg_ref[1]
    u = jnp.maximum(
        jnp.dot(h_ref[...], w1h_ref[...], preferred_element_type=_f32, precision=_HI)
        + jnp.dot(agg, w1a_ref[...], preferred_element_type=_f32, precision=_HI)
        + b1_ref[...], 0.0)
    hn_ref[...] = jnp.dot(u, w2_ref[...], preferred_element_type=_f32, precision=_HI) \
        + b2_ref[...]


def _prelu(x, a):
    return jnp.maximum(x, 0.0) + a * jnp.minimum(x, 0.0)


def _readout_body(h_ref, b_ref, spw_ref, spb_ref, spa_ref,
                  w1_ref, b1_ref, a1_ref, w2_ref, b2_ref, a2_ref,
                  w3_ref, b3_ref, out_ref, acc_ref):
    m = pl.program_id(0)
    onehot = (b_ref[...] == lax.broadcasted_iota(jnp.int32, (1, NG), 1)
              ).astype(_f32)
    part = lax.dot_general(onehot, h_ref[...], (((0,), (0,)), ((), ())),
                           preferred_element_type=_f32, precision=_HI)

    @pl.when(m == 0)
    def _():
        acc_ref[...] = part

    @pl.when(m > 0)
    def _():
        acc_ref[...] += part

    @pl.when(m == N // MB - 1)
    def _():
        r = jnp.dot(acc_ref[...], spw_ref[...], preferred_element_type=_f32, precision=_HI) \
            + spb_ref[...]
        r = _prelu(r, spa_ref[0, 0])
        p = jnp.dot(r, w1_ref[...], preferred_element_type=_f32, precision=_HI) + b1_ref[...]
        p = _prelu(p, a1_ref[0, 0])
        p = jnp.dot(p, w2_ref[...], preferred_element_type=_f32, precision=_HI) + b2_ref[...]
        p = _prelu(p, a2_ref[0, 0])
        out_ref[...] = jnp.dot(p, w3_ref[...], preferred_element_type=_f32, precision=_HI) \
            + b3_ref[...]


def _sc_edge_body(i, hw_hbm, ew_hbm, src_hbm, dst_hbm, z_hbm, agg_hbm,
                  src_v, dst_v, gbuf0, gbuf1, ebuf0, ebuf1, mbuf, agg_sh,
                  gsem0, gsem1, esem0, esem1):
    c = lax.axis_index("c")
    s = lax.axis_index("s")
    wid = s * NC + c
    ebase = wid * EPW

    @pl.when(s < NS - 1)
    def _():
        pltpu.sync_copy(z_hbm.at[pl.ds(s * RPT, RPT)],
                        agg_sh.at[pl.ds(s * RPT, RPT)])

    @pl.when(s == NS - 1)
    def _():
        pltpu.sync_copy(z_hbm.at[pl.ds((NS - 1) * RPT, RPT_LAST)],
                        agg_sh.at[pl.ds((NS - 1) * RPT, RPT_LAST)])

    plsc.subcore_barrier()

    def start(cc, j, gb, eb, gs, es):
        pltpu.async_copy(hw_hbm.at[src_v.at[j]], gb, gs)
        pltpu.async_copy(
            ew_hbm.at[pl.ds(ebase + (cc * CS + j) * SLAB, SLAB)],
            eb, es)

    def finish(cc, j, gb, eb, gs, es):
        pltpu.make_async_copy(hw_hbm.at[src_v.at[j]], gb, gs).wait()
        pltpu.make_async_copy(
            ew_hbm.at[pl.ds(ebase + (cc * CS + j) * SLAB, SLAB)],
            eb, es).wait()

        def row(r, c2):
            for k in range(HW // 16):
                ei = eb[r, pl.ds(16 * k, 16)]
                lo = jnp.maximum(
                    gb[r, pl.ds(16 * k, 16)]
                    + lax.bitcast_convert_type(ei << 16, _f32), 0.0)
                hi = jnp.maximum(
                    gb[r, pl.ds(HW + 16 * k, 16)]
                    + lax.bitcast_convert_type(ei & _HIMASK, _f32), 0.0)
                mbuf[r, pl.ds(16 * k, 16)] = lo
                mbuf[r, pl.ds(HW + 16 * k, 16)] = hi
            return c2

        lax.fori_loop(0, SLAB, row, 0)
        pltpu.sync_copy(mbuf, agg_sh.at[dst_v.at[j]], add=True)

    def chunk(cc, carry):
        pltpu.sync_copy(src_hbm.at[wid].at[cc], src_v)
        pltpu.sync_copy(dst_hbm.at[wid].at[cc], dst_v)
        start(cc, 0, gbuf0, ebuf0, gsem0, esem0)

        def pair(jj, c1):
            j0 = 2 * jj
            start(cc, j0 + 1, gbuf1, ebuf1, gsem1, esem1)
            finish(cc, j0, gbuf0, ebuf0, gsem0, esem0)

            @pl.when(jj < CS // 2 - 1)
            def _():
                start(cc, j0 + 2, gbuf0, ebuf0, gsem0, esem0)

            finish(cc, j0 + 1, gbuf1, ebuf1, gsem1, esem1)
            return c1

        lax.fori_loop(0, CS // 2, pair, 0)
        return carry

    lax.fori_loop(0, NCH, chunk, 0)
    plsc.subcore_barrier()

    @pl.when(s < NS - 1)
    def _():
        pltpu.sync_copy(agg_sh.at[pl.ds(s * RPT, RPT)],
                        agg_hbm.at[c].at[pl.ds(s * RPT, RPT)])

    @pl.when(s == NS - 1)
    def _():
        pltpu.sync_copy(agg_sh.at[pl.ds((NS - 1) * RPT, RPT_LAST)],
                        agg_hbm.at[c].at[pl.ds((NS - 1) * RPT, RPT_LAST)])


def _sc_aggregate(i, hw, ew, srcg, dstg, zeros):
    mesh = plsc.VectorSubcoreMesh(core_axis_name="c", subcore_axis_name="s")
    return pl.kernel(
        functools.partial(_sc_edge_body, i),
        out_type=jax.ShapeDtypeStruct((NC, N, H), _f32),
        mesh=mesh,
        scratch_types=[
            pltpu.VMEM((CS, SLAB), jnp.int32),
            pltpu.VMEM((CS, SLAB), jnp.int32),
            pltpu.VMEM((SLAB, H), _f32),
            pltpu.VMEM((SLAB, H), _f32),
            pltpu.VMEM((SLAB, HW), jnp.int32),
            pltpu.VMEM((SLAB, HW), jnp.int32),
            pltpu.VMEM((SLAB, H), _f32),
            pltpu.VMEM_SHARED((N, H), _f32),
            pltpu.SemaphoreType.DMA,
            pltpu.SemaphoreType.DMA,
            pltpu.SemaphoreType.DMA,
            pltpu.SemaphoreType.DMA,
        ],
    )(hw, ew, srcg, dstg, zeros)


def _full(shape):
    return pl.BlockSpec(shape, lambda *_: tuple(0 for _ in shape))


def kernel(x, edge_index, edge_attr, batch,
           pn_W, pn_b, pe_W, pe_b,
           msg_W, msg_b, upd_W1, upd_b1, upd_W2, upd_b2,
           sp_W, sp_b, sp_a,
           pr_W1, pr_b1, pr_a1, pr_W2, pr_b2, pr_a2, pr_W3, pr_b3):
    srcg = edge_index[0].reshape(NW, NCH, CS, SLAB)
    dstg = edge_index[1].reshape(NW, NCH, CS, SLAB)
    zeros = jnp.zeros((N, H), _f32)

    wt = msg_W[:, :H, :]
    wb = msg_W[:, H:, :]
    w1h = upd_W1[:, :H, :]
    w1a = upd_W1[:, H:, :]
    pn_b2 = pn_b.reshape(1, H)
    pe_b2 = pe_b.reshape(1, H)
    msg_b2 = msg_b.reshape(D, 1, H)
    b1_2 = upd_b1.reshape(D, 1, 2 * H)
    b2_2 = upd_b2.reshape(D, 1, H)

    nm = N // MB

    h, hw = pl.pallas_call(
        _init_body,
        grid=(nm,),
        in_specs=[
            pl.BlockSpec((MB, F), lambda m: (m, 0)),
            _full((F, H)), _full((1, H)), _full((H, H)), _full((H, H)),
            _full((1, H)), _full((1, H)),
        ],
        out_specs=[pl.BlockSpec((MB, H), lambda m: (m, 0)),
                   pl.BlockSpec((MB, H), lambda m: (m, 0))],
        out_shape=[jax.ShapeDtypeStruct((N, H), _f32),
                   jax.ShapeDtypeStruct((N, H), _f32)],
    )(x, pn_W, pn_b2, wt[0], wb[0], pe_b2, msg_b2[0])

    ne = E // EB
    ews = pl.pallas_call(
        _ew_body,
        grid=(ne,),
        in_specs=[
            pl.BlockSpec((EB, EF), lambda e: (e, 0)),
            pl.BlockSpec((EF, H), lambda e: (0, 0)),
            pl.BlockSpec((D, H, H), lambda e: (0, 0, 0)),
        ],
        out_specs=[pl.BlockSpec((EB, HW), lambda e: (e, 0))] * D,
        out_shape=[jax.ShapeDtypeStruct((E, HW), jnp.int32)] * D,
    )(edge_attr, pe_W, wb)

    for i in range(D):
        aggp = _sc_aggregate(i, hw, ews[i], srcg, dstg, zeros)
        if i < D - 1:
            h, hw = pl.pallas_call(
                _upd_body,
                grid=(nm,),
                in_specs=[
                    pl.BlockSpec((MB, H), lambda m: (m, 0)),
                    pl.BlockSpec((NC, MB, H), lambda m: (0, m, 0)),
                    _full((H, 2 * H)), _full((H, 2 * H)), _full((1, 2 * H)),
                    _full((2 * H, H)), _full((1, H)),
                    _full((H, H)), _full((H, H)), _full((1, H)),
                    _full((1, H)),
                ],
                out_specs=[pl.BlockSpec((MB, H), lambda m: (m, 0)),
                           pl.BlockSpec((MB, H), lambda m: (m, 0))],
                out_shape=[jax.ShapeDtypeStruct((N, H), _f32),
                           jax.ShapeDtypeStruct((N, H), _f32)],
            )(h, aggp, w1h[i], w1a[i], b1_2[i], upd_W2[i], b2_2[i],
              wt[i + 1], wb[i + 1], pe_b2, msg_b2[i + 1])
        else:
            h = pl.pallas_call(
                _upd_last_body,
                grid=(nm,),
                in_specs=[
                    pl.BlockSpec((MB, H), lambda m: (m, 0)),
                    pl.BlockSpec((NC, MB, H), lambda m: (0, m, 0)),
                    _full((H, 2 * H)), _full((H, 2 * H)), _full((1, 2 * H)),
                    _full((2 * H, H)), _full((1, H)),
                ],
                out_specs=pl.BlockSpec((MB, H), lambda m: (m, 0)),
                out_shape=jax.ShapeDtypeStruct((N, H), _f32),
            )(h, aggp, w1h[i], w1a[i], b1_2[i], upd_W2[i], b2_2[i])

    w3p = jnp.pad(pr_W3, ((0, 0), (0, F - pr_W3.shape[1])))
    out_p = pl.pallas_call(
        _readout_body,
        grid=(nm,),
        in_specs=[
            pl.BlockSpec((MB, H), lambda m: (m, 0)),
            pl.BlockSpec((MB, 1), lambda m: (m, 0)),
            _full((H, 256)), _full((1, 256)), _full((1, 1)),
            _full((256, 256)), _full((1, 256)), _full((1, 1)),
            _full((256, 256)), _full((1, 256)), _full((1, 1)),
            _full((256, F)), _full((1, 1)),
        ],
        out_specs=pl.BlockSpec((NG, F), lambda m: (0, 0)),
        out_shape=jax.ShapeDtypeStruct((NG, F), _f32),
        scratch_shapes=[pltpu.VMEM((NG, H), _f32)],
    )(h, batch.reshape(N, 1), sp_W, sp_b.reshape(1, 256),
      sp_a.reshape(1, 1).astype(_f32),
      pr_W1, pr_b1.reshape(1, 256), pr_a1.reshape(1, 1).astype(_f32),
      pr_W2, pr_b2.reshape(1, 256), pr_a2.reshape(1, 1).astype(_f32),
      w3p, pr_b3.reshape(1, 1))
    return out_p[:, :pr_W3.shape[1]]
